# baseline (speedup 1.0000x reference)
import functools

import jax
import jax.numpy as jnp
from jax import lax
from jax.experimental import pallas as pl
from jax.experimental.pallas import tpu as pltpu
from jax.experimental.pallas import tpu_sc as plsc

N = 10000
E = 320000
D = 128
S = 9

NC = 2
NS = 16
L = 16
NW = NC * NS

EPW = E // NW
CHUNK = 80
NSTEPS = EPW // CHUNK
RPS = 624
REMR = N - NS * RPS
ZROWS = 156


SELQ = 2000


def _sc_agg_body(table, srcs, sels, dsts, zrows, out,
                 gidx_v, selq_v, dst0_v, dst1_v, rows0_v, rows1_v, acc_sh,
                 sem0, sem1, semd0, semd1):
    cid = lax.axis_index("c")
    sid = lax.axis_index("s")
    wid = cid * NS + sid
    ebase = wid * EPW

    pltpu.sync_copy(zrows, acc_sh.at[pl.ds(sid * RPS, RPS)])

    @pl.when(sid == 0)
    def _zero_rem():
        pltpu.sync_copy(zrows.at[pl.ds(0, REMR)],
                        acc_sh.at[pl.ds(NS * RPS, REMR)])

    pltpu.sync_copy(srcs.at[pl.ds(ebase, EPW)], gidx_v)
    for q in range(EPW // SELQ):
        pltpu.sync_copy(sels.at[pl.ds(ebase + q * SELQ, SELQ)], selq_v)

        def _gix(k, carry):
            sl = pl.ds(q * SELQ + k * L, L)
            gidx_v[sl] = selq_v[pl.ds(k * L, L)] * N + gidx_v[sl]
            return carry
        lax.fori_loop(0, SELQ // L, _gix, 0)
    plsc.subcore_barrier()

    def _gather(c, buf, sem):
        return pltpu.make_async_copy(
            table.at[gidx_v.at[pl.ds(c * CHUNK, CHUNK)]], buf, sem)

    def _dstcp(c, buf, sem):
        return pltpu.make_async_copy(
            dsts.at[pl.ds(ebase + c * CHUNK, CHUNK)], buf, sem)

    def _scatter(buf, dbuf):
        pltpu.sync_copy(buf, acc_sh.at[dbuf], add=True)

    _gather(0, rows0_v, sem0).start()
    _dstcp(0, dst0_v, semd0).start()

    def _step(k, carry):
        c0 = 2 * k
        _gather(c0, rows0_v, sem0).wait()
        _dstcp(c0, dst0_v, semd0).wait()
        _gather(c0 + 1, rows1_v, sem1).start()
        _dstcp(c0 + 1, dst1_v, semd1).start()
        _scatter(rows0_v, dst0_v)
        _gather(c0 + 2, rows0_v, sem0).start()
        _dstcp(c0 + 2, dst0_v, semd0).start()
        _gather(c0 + 1, rows1_v, sem1).wait()
        _dstcp(c0 + 1, dst1_v, semd1).wait()
        _scatter(rows1_v, dst1_v)
        return carry
    lax.fori_loop(0, (NSTEPS - 1) // 2, _step, 0)

    _gather(NSTEPS - 1, rows0_v, sem0).wait()
    _dstcp(NSTEPS - 1, dst0_v, semd0).wait()
    _scatter(rows0_v, dst0_v)

    plsc.subcore_barrier()
    pltpu.sync_copy(acc_sh.at[pl.ds(sid * RPS, RPS)],
                    out.at[cid, pl.ds(sid * RPS, RPS)])

    @pl.when(sid == 0)
    def _out_rem():
        pltpu.sync_copy(acc_sh.at[pl.ds(NS * RPS, REMR)],
                        out.at[cid, pl.ds(NS * RPS, REMR)])


def _sc_agg(table, srcs, sels, dsts, zrows):
    mesh = plsc.VectorSubcoreMesh(core_axis_name="c", subcore_axis_name="s")
    f = functools.partial(
        pl.kernel,
        mesh=mesh,
        out_type=jax.ShapeDtypeStruct((NC, N, D), jnp.float32),
        scratch_types=[
            pltpu.VMEM((EPW,), jnp.int32),
            pltpu.VMEM((SELQ,), jnp.int32),
            pltpu.VMEM((CHUNK,), jnp.int32),
            pltpu.VMEM((CHUNK,), jnp.int32),
            pltpu.VMEM((CHUNK, D), jnp.float32),
            pltpu.VMEM((CHUNK, D), jnp.float32),
            pltpu.VMEM_SHARED((N, D), jnp.float32),
            pltpu.SemaphoreType.DMA,
            pltpu.SemaphoreType.DMA,
            pltpu.SemaphoreType.DMA,
            pltpu.SemaphoreType.DMA,
        ],
    )(_sc_agg_body)
    return f(table, srcs, sels, dsts, zrows)


BM = 400
NB = N // BM


def _elu(x):
    return jnp.where(x > 0, x, jnp.exp(jnp.minimum(x, 0.0)) - 1.0)


def _mm_body(x_ref, w_ref, o_ref):
    o_ref[...] = jnp.dot(x_ref[...], w_ref[0],
                         preferred_element_type=jnp.float32)


def _mm_table(h, W):
    return pl.pallas_call(
        _mm_body,
        grid=(NB, S),
        in_specs=[
            pl.BlockSpec((BM, D), lambda i, s: (i, 0)),
            pl.BlockSpec((1, D, D), lambda i, s: (s, 0, 0)),
        ],
        out_specs=pl.BlockSpec((BM, D), lambda i, s: (s * NB + i, 0)),
        out_shape=jax.ShapeDtypeStruct((S * N, D), jnp.float32),
    )(h, W)


def _bn_mm_body(parts_ref, b_ref, g_ref, be_ref, w_ref, o_ref,
                stat_ref, h_ref):
    p = pl.program_id(0)
    i = pl.program_id(1)
    rows = pl.ds(i * BM, BM)

    @pl.when(p == 0)
    def _acc():
        a = parts_ref[0] + parts_ref[1] + b_ref[...]
        h_ref[rows, :] = a

        @pl.when(i == 0)
        def _init():
            stat_ref[...] = jnp.zeros_like(stat_ref)
        stat_ref[0:1, :] += jnp.sum(a, axis=0, keepdims=True)
        stat_ref[1:2, :] += jnp.sum(a * a, axis=0, keepdims=True)

    @pl.when(p == 1)
    def _norm_mm():
        m = stat_ref[0:1, :] / N
        v = stat_ref[1:2, :] / N - m * m
        h = _elu(g_ref[...] * (h_ref[rows, :] - m) * lax.rsqrt(v + 1e-5)
                 + be_ref[...])
        h_ref[rows, :] = h
        o_ref[...] = jnp.dot(h, w_ref[0], preferred_element_type=jnp.float32)

    @pl.when(p >= 2)
    def _mm():
        o_ref[...] = jnp.dot(h_ref[rows, :], w_ref[0],
                             preferred_element_type=jnp.float32)


def _bn_mm(parts, b, g, be, W):
    vspec = pl.BlockSpec((1, D), lambda p, i: (0, 0))
    return pl.pallas_call(
        _bn_mm_body,
        grid=(1 + S, NB),
        in_specs=[
            pl.BlockSpec((NC, BM, D),
                         lambda p, i: (0, jnp.where(p == 0, i, 0), 0)),
            vspec, vspec, vspec,
            pl.BlockSpec((1, D, D),
                         lambda p, i: (jnp.maximum(p - 1, 0), 0, 0)),
        ],
        out_specs=pl.BlockSpec(
            (BM, D), lambda p, i: (jnp.maximum(p - 1, 0) * NB + i, 0)),
        out_shape=jax.ShapeDtypeStruct((S * N, D), jnp.float32),
        scratch_shapes=[
            pltpu.VMEM((2, D), jnp.float32),
            pltpu.VMEM((N, D), jnp.float32),
        ],
    )(parts, b.reshape(1, D), g.reshape(1, D), be.reshape(1, D), W)


def _final_body(parts_ref, b2_ref, g2_ref, be2_ref, x_ref, w3_ref, b3_ref,
                g3_ref, be3_ref, o_ref, s2_ref, s3_ref, y_ref):
    p = pl.program_id(0)
    i = pl.program_id(1)
    rows = pl.ds(i * BM, BM)

    @pl.when(p == 0)
    def _acc2():
        a = parts_ref[0] + parts_ref[1] + b2_ref[...]
        y_ref[rows, :] = a

        @pl.when(i == 0)
        def _init():
            s2_ref[...] = jnp.zeros_like(s2_ref)
            s3_ref[...] = jnp.zeros_like(s3_ref)
        s2_ref[0:1, :] += jnp.sum(a, axis=0, keepdims=True)
        s2_ref[1:2, :] += jnp.sum(a * a, axis=0, keepdims=True)

    @pl.when(p == 1)
    def _mk_y():
        m = s2_ref[0:1, :] / N
        v = s2_ref[1:2, :] / N - m * m
        h2 = _elu(g2_ref[...] * (y_ref[rows, :] - m) * lax.rsqrt(v + 1e-5)
                  + be2_ref[...])
        y = h2 + jnp.dot(x_ref[...], w3_ref[...],
                         preferred_element_type=jnp.float32) + b3_ref[...]
        y_ref[rows, :] = y
        s3_ref[0:1, :] += jnp.sum(y, axis=0, keepdims=True)
        s3_ref[1:2, :] += jnp.sum(y * y, axis=0, keepdims=True)

    @pl.when(p == 2)
    def _apply():
        m = s3_ref[0:1, :] / N
        v = s3_ref[1:2, :] / N - m * m
        o_ref[...] = _elu(g3_ref[...] * (y_ref[rows, :] - m)
                          * lax.rsqrt(v + 1e-5) + be3_ref[...])


def _final(parts, b2, g2, be2, x, W3, b3, g3, be3):
    vecs = [v.reshape(1, D) for v in (b2, g2, be2, b3, g3, be3)]
    vspec = pl.BlockSpec((1, D), lambda p, i: (0, 0))
    return pl.pallas_call(
        _final_body,
        grid=(3, NB),
        in_specs=[
            pl.BlockSpec((NC, BM, D),
                         lambda p, i: (0, jnp.where(p == 0, i, 0), 0)),
            vspec, vspec, vspec,
            pl.BlockSpec((BM, D), lambda p, i: (jnp.where(p == 1, i, 0), 0)),
            pl.BlockSpec((D, D), lambda p, i: (0, 0)),
            vspec, vspec, vspec,
        ],
        out_specs=pl.BlockSpec(
            (BM, D), lambda p, i: (jnp.where(p == 2, i, 0), 0)),
        out_shape=jax.ShapeDtypeStruct((N, D), jnp.float32),
        scratch_shapes=[
            pltpu.VMEM((2, D), jnp.float32),
            pltpu.VMEM((2, D), jnp.float32),
            pltpu.VMEM((N, D), jnp.float32),
        ],
    )(parts, vecs[0], vecs[1], vecs[2], x, W3, vecs[3], vecs[4], vecs[5])


def kernel(x, edge_index, selections, W1, b1, g1, be1, W2, b2, g2, be2,
           W3, b3, g3, be3):
    src = edge_index[0].astype(jnp.int32)
    dst = edge_index[1].astype(jnp.int32)
    sel = selections.astype(jnp.int32)

    zrows = jnp.zeros((RPS, D), jnp.float32)

    table1 = _mm_table(x, W1)
    parts1 = _sc_agg(table1, src, sel, dst, zrows)
    table2 = _bn_mm(parts1, b1, g1, be1, W2)
    parts2 = _sc_agg(table2, src, sel, dst, zrows)
    return _final(parts2, b2, g2, be2, x, W3, b3, g3, be3)


# speedup vs baseline: 1.4744x; 1.4744x over previous
import functools

import jax
import jax.numpy as jnp
from jax import lax
from jax.experimental import pallas as pl
from jax.experimental.pallas import tpu as pltpu
from jax.experimental.pallas import tpu_sc as plsc

N = 10000
E = 320000
D = 128
S = 9

NC = 2
NS = 16
L = 16
NW = NC * NS

EPW = E // NW
CHUNK = 80
NSTEPS = EPW // CHUNK
RPS = 624
REMR = N - NS * RPS
ZROWS = 156


SELQ = 2000


def _sc_agg_body(table, srcs, sels, dsts, zrows, out,
                 gidx_v, selq_v, dst0_v, dst1_v, rows0_v, rows1_v, acc_sh,
                 sem0, sem1, semd0, semd1):
    cid = lax.axis_index("c")
    sid = lax.axis_index("s")
    wid = cid * NS + sid
    ebase = wid * EPW

    pltpu.sync_copy(zrows, acc_sh.at[pl.ds(sid * RPS, RPS)])

    @pl.when(sid == 0)
    def _zero_rem():
        pltpu.sync_copy(zrows.at[pl.ds(0, REMR)],
                        acc_sh.at[pl.ds(NS * RPS, REMR)])

    pltpu.sync_copy(srcs.at[pl.ds(ebase, EPW)], gidx_v)
    for q in range(EPW // SELQ):
        pltpu.sync_copy(sels.at[pl.ds(ebase + q * SELQ, SELQ)], selq_v)

        def _gix(k, carry):
            sl = pl.ds(q * SELQ + k * L, L)
            gidx_v[sl] = selq_v[pl.ds(k * L, L)] * N + gidx_v[sl]
            return carry
        lax.fori_loop(0, SELQ // L, _gix, 0)
    plsc.subcore_barrier()

    def _gather(c, buf, sem):
        return pltpu.make_async_copy(
            table.at[gidx_v.at[pl.ds(c * CHUNK, CHUNK)]], buf, sem)

    def _dstcp(c, buf, sem):
        return pltpu.make_async_copy(
            dsts.at[pl.ds(ebase + c * CHUNK, CHUNK)], buf, sem)

    def _scatter(buf, dbuf):
        pltpu.sync_copy(buf, acc_sh.at[dbuf], add=True)

    _gather(0, rows0_v, sem0).start()
    _dstcp(0, dst0_v, semd0).start()

    def _step(k, carry):
        c0 = 2 * k
        _gather(c0, rows0_v, sem0).wait()
        _dstcp(c0, dst0_v, semd0).wait()
        _gather(c0 + 1, rows1_v, sem1).start()
        _dstcp(c0 + 1, dst1_v, semd1).start()
        _scatter(rows0_v, dst0_v)
        _gather(c0 + 2, rows0_v, sem0).start()
        _dstcp(c0 + 2, dst0_v, semd0).start()
        _gather(c0 + 1, rows1_v, sem1).wait()
        _dstcp(c0 + 1, dst1_v, semd1).wait()
        _scatter(rows1_v, dst1_v)
        return carry
    lax.fori_loop(0, (NSTEPS - 1) // 2, _step, 0)

    _gather(NSTEPS - 1, rows0_v, sem0).wait()
    _dstcp(NSTEPS - 1, dst0_v, semd0).wait()
    _scatter(rows0_v, dst0_v)

    plsc.subcore_barrier()
    pltpu.sync_copy(acc_sh.at[pl.ds(sid * RPS, RPS)],
                    out.at[cid, pl.ds(sid * RPS, RPS)])

    @pl.when(sid == 0)
    def _out_rem():
        pltpu.sync_copy(acc_sh.at[pl.ds(NS * RPS, REMR)],
                        out.at[cid, pl.ds(NS * RPS, REMR)])


def _sc_agg(table, srcs, sels, dsts, zrows):
    mesh = plsc.VectorSubcoreMesh(core_axis_name="c", subcore_axis_name="s")
    f = functools.partial(
        pl.kernel,
        mesh=mesh,
        out_type=jax.ShapeDtypeStruct((NC, N, D), jnp.float32),
        scratch_types=[
            pltpu.VMEM((EPW,), jnp.int32),
            pltpu.VMEM((SELQ,), jnp.int32),
            pltpu.VMEM((CHUNK,), jnp.int32),
            pltpu.VMEM((CHUNK,), jnp.int32),
            pltpu.VMEM((CHUNK, D), jnp.float32),
            pltpu.VMEM((CHUNK, D), jnp.float32),
            pltpu.VMEM_SHARED((N, D), jnp.float32),
            pltpu.SemaphoreType.DMA,
            pltpu.SemaphoreType.DMA,
            pltpu.SemaphoreType.DMA,
            pltpu.SemaphoreType.DMA,
        ],
    )(_sc_agg_body)
    return f(table, srcs, sels, dsts, zrows)


BM = 400
NB = N // BM


def _elu(x):
    return jnp.where(x > 0, x, jnp.exp(jnp.minimum(x, 0.0)) - 1.0)


def _mm_body(x_ref, w_ref, o_ref):
    acc = jnp.dot(x_ref[...], w_ref[...], preferred_element_type=jnp.float32)
    for s in range(S):
        o_ref[s] = acc[:, s * D:(s + 1) * D]


def _mm_table(h, Wr):
    return pl.pallas_call(
        _mm_body,
        grid=(NB,),
        in_specs=[
            pl.BlockSpec((BM, D), lambda i: (i, 0)),
            pl.BlockSpec((D, S * D), lambda i: (0, 0)),
        ],
        out_specs=pl.BlockSpec((S, BM, D), lambda i: (0, i, 0)),
        out_shape=jax.ShapeDtypeStruct((S, N, D), jnp.float32),
    )(h, Wr)


def _bn_mm_body(parts_ref, b_ref, g_ref, be_ref, w_ref, o_ref,
                stat_ref, h_ref):
    p = pl.program_id(0)
    i = pl.program_id(1)
    rows = pl.ds(i * BM, BM)

    @pl.when(p == 0)
    def _acc():
        a = parts_ref[0] + parts_ref[1] + b_ref[...]
        h_ref[rows, :] = a

        @pl.when(i == 0)
        def _init():
            stat_ref[...] = jnp.zeros_like(stat_ref)
        stat_ref[0:1, :] += jnp.sum(a, axis=0, keepdims=True)
        stat_ref[1:2, :] += jnp.sum(a * a, axis=0, keepdims=True)

    @pl.when(p == 1)
    def _norm_mm():
        m = stat_ref[0:1, :] / N
        v = stat_ref[1:2, :] / N - m * m
        h = _elu(g_ref[...] * (h_ref[rows, :] - m) * lax.rsqrt(v + 1e-5)
                 + be_ref[...])
        h_ref[rows, :] = h
        acc = jnp.dot(h, w_ref[...], preferred_element_type=jnp.float32)
        for s in range(S):
            o_ref[s] = acc[:, s * D:(s + 1) * D]


def _bn_mm(parts, b, g, be, Wr):
    vspec = pl.BlockSpec((1, D), lambda p, i: (0, 0))
    return pl.pallas_call(
        _bn_mm_body,
        grid=(2, NB),
        in_specs=[
            pl.BlockSpec((NC, BM, D),
                         lambda p, i: (0, jnp.where(p == 0, i, 0), 0)),
            vspec, vspec, vspec,
            pl.BlockSpec((D, S * D), lambda p, i: (0, 0)),
        ],
        out_specs=pl.BlockSpec(
            (S, BM, D), lambda p, i: (0, jnp.where(p == 1, i, 0), 0)),
        out_shape=jax.ShapeDtypeStruct((S, N, D), jnp.float32),
        scratch_shapes=[
            pltpu.VMEM((2, D), jnp.float32),
            pltpu.VMEM((N, D), jnp.float32),
        ],
    )(parts, b.reshape(1, D), g.reshape(1, D), be.reshape(1, D), Wr)


def _final_body(parts_ref, b2_ref, g2_ref, be2_ref, x_ref, w3_ref, b3_ref,
                g3_ref, be3_ref, o_ref, s2_ref, s3_ref, y_ref):
    p = pl.program_id(0)
    i = pl.program_id(1)
    rows = pl.ds(i * BM, BM)

    @pl.when(p == 0)
    def _acc2():
        a = parts_ref[0] + parts_ref[1] + b2_ref[...]
        y_ref[rows, :] = a

        @pl.when(i == 0)
        def _init():
            s2_ref[...] = jnp.zeros_like(s2_ref)
            s3_ref[...] = jnp.zeros_like(s3_ref)
        s2_ref[0:1, :] += jnp.sum(a, axis=0, keepdims=True)
        s2_ref[1:2, :] += jnp.sum(a * a, axis=0, keepdims=True)

    @pl.when(p == 1)
    def _mk_y():
        m = s2_ref[0:1, :] / N
        v = s2_ref[1:2, :] / N - m * m
        h2 = _elu(g2_ref[...] * (y_ref[rows, :] - m) * lax.rsqrt(v + 1e-5)
                  + be2_ref[...])
        y = h2 + jnp.dot(x_ref[...], w3_ref[...],
                         preferred_element_type=jnp.float32) + b3_ref[...]
        y_ref[rows, :] = y
        s3_ref[0:1, :] += jnp.sum(y, axis=0, keepdims=True)
        s3_ref[1:2, :] += jnp.sum(y * y, axis=0, keepdims=True)

    @pl.when(p == 2)
    def _apply():
        m = s3_ref[0:1, :] / N
        v = s3_ref[1:2, :] / N - m * m
        o_ref[...] = _elu(g3_ref[...] * (y_ref[rows, :] - m)
                          * lax.rsqrt(v + 1e-5) + be3_ref[...])


def _final(parts, b2, g2, be2, x, W3, b3, g3, be3):
    vecs = [v.reshape(1, D) for v in (b2, g2, be2, b3, g3, be3)]
    vspec = pl.BlockSpec((1, D), lambda p, i: (0, 0))
    return pl.pallas_call(
        _final_body,
        grid=(3, NB),
        in_specs=[
            pl.BlockSpec((NC, BM, D),
                         lambda p, i: (0, jnp.where(p == 0, i, 0), 0)),
            vspec, vspec, vspec,
            pl.BlockSpec((BM, D), lambda p, i: (jnp.where(p == 1, i, 0), 0)),
            pl.BlockSpec((D, D), lambda p, i: (0, 0)),
            vspec, vspec, vspec,
        ],
        out_specs=pl.BlockSpec(
            (BM, D), lambda p, i: (jnp.where(p == 2, i, 0), 0)),
        out_shape=jax.ShapeDtypeStruct((N, D), jnp.float32),
        scratch_shapes=[
            pltpu.VMEM((2, D), jnp.float32),
            pltpu.VMEM((2, D), jnp.float32),
            pltpu.VMEM((N, D), jnp.float32),
        ],
    )(parts, vecs[0], vecs[1], vecs[2], x, W3, vecs[3], vecs[4], vecs[5])


def kernel(x, edge_index, selections, W1, b1, g1, be1, W2, b2, g2, be2,
           W3, b3, g3, be3):
    src = edge_index[0].astype(jnp.int32)
    dst = edge_index[1].astype(jnp.int32)
    sel = selections.astype(jnp.int32)

    zrows = jnp.zeros((RPS, D), jnp.float32)

    Wr1 = W1.transpose(1, 0, 2).reshape(D, S * D)
    Wr2 = W2.transpose(1, 0, 2).reshape(D, S * D)

    table1 = _mm_table(x, Wr1).reshape(S * N, D)
    parts1 = _sc_agg(table1, src, sel, dst, zrows)
    table2 = _bn_mm(parts1, b1, g1, be1, Wr2).reshape(S * N, D)
    parts2 = _sc_agg(table2, src, sel, dst, zrows)
    return _final(parts2, b2, g2, be2, x, W3, b3, g3, be3)


# speedup vs baseline: 1.7083x; 1.1587x over previous
import functools

import jax
import jax.numpy as jnp
from jax import lax
from jax.experimental import pallas as pl
from jax.experimental.pallas import tpu as pltpu
from jax.experimental.pallas import tpu_sc as plsc

N = 10000
E = 320000
D = 128
S = 9

NC = 2
NS = 16
L = 16
NW = NC * NS

EPW = E // NW
CHUNK = 128
NSTEPS = EPW // CHUNK
REME = EPW - NSTEPS * CHUNK
RPS = 624
REMR = N - NS * RPS
ZROWS = 156


def _sc_agg_body(table, gidx, dsts, zrows, out,
                 gidx_v, dst0_v, dst1_v, dstr_v, rows0_v, rows1_v, acc_sh,
                 sem0, sem1, semd0, semd1):
    cid = lax.axis_index("c")
    sid = lax.axis_index("s")
    wid = cid * NS + sid
    ebase = wid * EPW

    pltpu.sync_copy(zrows, acc_sh.at[pl.ds(sid * RPS, RPS)])

    @pl.when(sid == 0)
    def _zero_rem():
        pltpu.sync_copy(zrows.at[pl.ds(0, REMR)],
                        acc_sh.at[pl.ds(NS * RPS, REMR)])

    pltpu.sync_copy(gidx.at[pl.ds(ebase, EPW)], gidx_v)
    plsc.subcore_barrier()

    def _gather(c, buf, sem):
        return pltpu.make_async_copy(
            table.at[gidx_v.at[pl.ds(c * CHUNK, CHUNK)]], buf, sem)

    def _dstcp(c, buf, sem):
        return pltpu.make_async_copy(
            dsts.at[pl.ds(ebase + c * CHUNK, CHUNK)], buf, sem)

    def _scatter(buf, dbuf):
        pltpu.sync_copy(buf, acc_sh.at[dbuf], add=True)

    _gather(0, rows0_v, sem0).start()
    _dstcp(0, dst0_v, semd0).start()

    def _step(k, carry):
        c0 = 2 * k
        _gather(c0, rows0_v, sem0).wait()
        _dstcp(c0, dst0_v, semd0).wait()
        _gather(c0 + 1, rows1_v, sem1).start()
        _dstcp(c0 + 1, dst1_v, semd1).start()
        _scatter(rows0_v, dst0_v)

        @pl.when(c0 + 2 < NSTEPS)
        def _pref():
            _gather(c0 + 2, rows0_v, sem0).start()
            _dstcp(c0 + 2, dst0_v, semd0).start()
        _gather(c0 + 1, rows1_v, sem1).wait()
        _dstcp(c0 + 1, dst1_v, semd1).wait()
        _scatter(rows1_v, dst1_v)
        return carry
    lax.fori_loop(0, NSTEPS // 2, _step, 0)

    rsl = pl.ds(0, REME)
    pltpu.sync_copy(dsts.at[pl.ds(ebase + NSTEPS * CHUNK, REME)], dstr_v)
    pltpu.async_copy(table.at[gidx_v.at[pl.ds(NSTEPS * CHUNK, REME)]],
                     rows0_v.at[rsl], sem0).wait()
    pltpu.sync_copy(rows0_v.at[rsl], acc_sh.at[dstr_v], add=True)

    plsc.subcore_barrier()
    pltpu.sync_copy(acc_sh.at[pl.ds(sid * RPS, RPS)],
                    out.at[cid, pl.ds(sid * RPS, RPS)])

    @pl.when(sid == 0)
    def _out_rem():
        pltpu.sync_copy(acc_sh.at[pl.ds(NS * RPS, REMR)],
                        out.at[cid, pl.ds(NS * RPS, REMR)])


def _sc_agg(table, gidx, dsts, zrows):
    mesh = plsc.VectorSubcoreMesh(core_axis_name="c", subcore_axis_name="s")
    f = functools.partial(
        pl.kernel,
        mesh=mesh,
        out_type=jax.ShapeDtypeStruct((NC, N, D), jnp.float32),
        scratch_types=[
            pltpu.VMEM((EPW,), jnp.int32),
            pltpu.VMEM((CHUNK,), jnp.int32),
            pltpu.VMEM((CHUNK,), jnp.int32),
            pltpu.VMEM((REME,), jnp.int32),
            pltpu.VMEM((CHUNK, D), jnp.float32),
            pltpu.VMEM((CHUNK, D), jnp.float32),
            pltpu.VMEM_SHARED((N, D), jnp.float32),
            pltpu.SemaphoreType.DMA,
            pltpu.SemaphoreType.DMA,
            pltpu.SemaphoreType.DMA,
            pltpu.SemaphoreType.DMA,
        ],
    )(_sc_agg_body)
    return f(table, gidx, dsts, zrows)


BM = 400
NB = N // BM


def _elu(x):
    return jnp.where(x > 0, x, jnp.exp(jnp.minimum(x, 0.0)) - 1.0)


def _mm_body(x_ref, w_ref, o_ref):
    acc = jnp.dot(x_ref[...], w_ref[...], preferred_element_type=jnp.float32)
    for s in range(S):
        o_ref[s] = acc[:, s * D:(s + 1) * D]


def _mm_table(h, Wr):
    return pl.pallas_call(
        _mm_body,
        grid=(NB,),
        in_specs=[
            pl.BlockSpec((BM, D), lambda i: (i, 0)),
            pl.BlockSpec((D, S * D), lambda i: (0, 0)),
        ],
        out_specs=pl.BlockSpec((S, BM, D), lambda i: (0, i, 0)),
        out_shape=jax.ShapeDtypeStruct((S, N, D), jnp.float32),
    )(h, Wr)


def _bn_mm_body(parts_ref, b_ref, g_ref, be_ref, w_ref, o_ref,
                stat_ref, h_ref):
    p = pl.program_id(0)
    i = pl.program_id(1)
    rows = pl.ds(i * BM, BM)

    @pl.when(p == 0)
    def _acc():
        a = parts_ref[0] + parts_ref[1] + b_ref[...]
        h_ref[rows, :] = a

        @pl.when(i == 0)
        def _init():
            stat_ref[...] = jnp.zeros_like(stat_ref)
        stat_ref[0:1, :] += jnp.sum(a, axis=0, keepdims=True)
        stat_ref[1:2, :] += jnp.sum(a * a, axis=0, keepdims=True)

    @pl.when(p == 1)
    def _norm_mm():
        m = stat_ref[0:1, :] / N
        v = stat_ref[1:2, :] / N - m * m
        h = _elu(g_ref[...] * (h_ref[rows, :] - m) * lax.rsqrt(v + 1e-5)
                 + be_ref[...])
        h_ref[rows, :] = h
        acc = jnp.dot(h, w_ref[...], preferred_element_type=jnp.float32)
        for s in range(S):
            o_ref[s] = acc[:, s * D:(s + 1) * D]


def _bn_mm(parts, b, g, be, Wr):
    vspec = pl.BlockSpec((1, D), lambda p, i: (0, 0))
    return pl.pallas_call(
        _bn_mm_body,
        grid=(2, NB),
        in_specs=[
            pl.BlockSpec((NC, BM, D),
                         lambda p, i: (0, jnp.where(p == 0, i, 0), 0)),
            vspec, vspec, vspec,
            pl.BlockSpec((D, S * D), lambda p, i: (0, 0)),
        ],
        out_specs=pl.BlockSpec(
            (S, BM, D), lambda p, i: (0, jnp.where(p == 1, i, 0), 0)),
        out_shape=jax.ShapeDtypeStruct((S, N, D), jnp.float32),
        scratch_shapes=[
            pltpu.VMEM((2, D), jnp.float32),
            pltpu.VMEM((N, D), jnp.float32),
        ],
    )(parts, b.reshape(1, D), g.reshape(1, D), be.reshape(1, D), Wr)


BF = 2000
NBF = N // BF


def _final_body(parts_ref, b2_ref, g2_ref, be2_ref, x_ref, w3_ref, b3_ref,
                g3_ref, be3_ref, o_ref, s2_ref, s3_ref, y_ref):
    p = pl.program_id(0)
    i = pl.program_id(1)
    rows = pl.ds(i * BF, BF)

    @pl.when(p == 0)
    def _acc2():
        a = parts_ref[0] + parts_ref[1] + b2_ref[...]
        y_ref[rows, :] = a

        @pl.when(i == 0)
        def _init():
            s2_ref[...] = jnp.zeros_like(s2_ref)
            s3_ref[...] = jnp.zeros_like(s3_ref)
        s2_ref[0:1, :] += jnp.sum(a, axis=0, keepdims=True)
        s2_ref[1:2, :] += jnp.sum(a * a, axis=0, keepdims=True)

    @pl.when(p == 1)
    def _mk_y():
        m = s2_ref[0:1, :] / N
        v = s2_ref[1:2, :] / N - m * m
        h2 = _elu(g2_ref[...] * (y_ref[rows, :] - m) * lax.rsqrt(v + 1e-5)
                  + be2_ref[...])
        y = h2 + jnp.dot(x_ref[...], w3_ref[...],
                         preferred_element_type=jnp.float32) + b3_ref[...]
        y_ref[rows, :] = y
        s3_ref[0:1, :] += jnp.sum(y, axis=0, keepdims=True)
        s3_ref[1:2, :] += jnp.sum(y * y, axis=0, keepdims=True)

    @pl.when(p == 2)
    def _apply():
        m = s3_ref[0:1, :] / N
        v = s3_ref[1:2, :] / N - m * m
        o_ref[...] = _elu(g3_ref[...] * (y_ref[rows, :] - m)
                          * lax.rsqrt(v + 1e-5) + be3_ref[...])


def _final(parts, b2, g2, be2, x, W3, b3, g3, be3):
    vecs = [v.reshape(1, D) for v in (b2, g2, be2, b3, g3, be3)]
    vspec = pl.BlockSpec((1, D), lambda p, i: (0, 0))
    return pl.pallas_call(
        _final_body,
        grid=(3, NBF),
        in_specs=[
            pl.BlockSpec((NC, BF, D),
                         lambda p, i: (0, jnp.where(p == 0, i, 0), 0)),
            vspec, vspec, vspec,
            pl.BlockSpec((BF, D), lambda p, i: (jnp.where(p == 1, i, 0), 0)),
            pl.BlockSpec((D, D), lambda p, i: (0, 0)),
            vspec, vspec, vspec,
        ],
        out_specs=pl.BlockSpec(
            (BF, D), lambda p, i: (jnp.where(p == 2, i, 0), 0)),
        out_shape=jax.ShapeDtypeStruct((N, D), jnp.float32),
        scratch_shapes=[
            pltpu.VMEM((2, D), jnp.float32),
            pltpu.VMEM((2, D), jnp.float32),
            pltpu.VMEM((N, D), jnp.float32),
        ],
    )(parts, vecs[0], vecs[1], vecs[2], x, W3, vecs[3], vecs[4], vecs[5])


def kernel(x, edge_index, selections, W1, b1, g1, be1, W2, b2, g2, be2,
           W3, b3, g3, be3):
    src = edge_index[0].astype(jnp.int32)
    dst = edge_index[1].astype(jnp.int32)
    sel = selections.astype(jnp.int32)
    gidx = sel * N + src

    zrows = jnp.zeros((RPS, D), jnp.float32)

    Wr1 = W1.transpose(1, 0, 2).reshape(D, S * D)
    Wr2 = W2.transpose(1, 0, 2).reshape(D, S * D)

    table1 = _mm_table(x, Wr1).reshape(S * N, D)
    parts1 = _sc_agg(table1, gidx, dst, zrows)
    table2 = _bn_mm(parts1, b1, g1, be1, Wr2).reshape(S * N, D)
    parts2 = _sc_agg(table2, gidx, dst, zrows)
    return _final(parts2, b2, g2, be2, x, W3, b3, g3, be3)


# speedup vs baseline: 1.8645x; 1.0914x over previous
import functools

import jax
import jax.numpy as jnp
from jax import lax
from jax.experimental import pallas as pl
from jax.experimental.pallas import tpu as pltpu
from jax.experimental.pallas import tpu_sc as plsc

N = 10000
E = 320000
D = 128
S = 9

NC = 2
NS = 16
L = 16
NW = NC * NS

EPW = E // NW
CHUNK = 128
NSTEPS = EPW // CHUNK
REME = EPW - NSTEPS * CHUNK
RPS = 624
REMR = N - NS * RPS
ZROWS = 156


def _sc_agg_body(table, gidx, dsts, zrows, out,
                 gidx_v, dst0_v, dst1_v, dstr_v, rows0_v, rows1_v, acc_sh,
                 sem0, sem1, semd0, semd1):
    cid = lax.axis_index("c")
    sid = lax.axis_index("s")
    wid = cid * NS + sid
    ebase = wid * EPW

    pltpu.sync_copy(gidx.at[pl.ds(ebase, EPW)], gidx_v)

    def _gather(c, buf, sem):
        return pltpu.make_async_copy(
            table.at[gidx_v.at[pl.ds(c * CHUNK, CHUNK)]], buf, sem)

    def _dstcp(c, buf, sem):
        return pltpu.make_async_copy(
            dsts.at[pl.ds(ebase + c * CHUNK, CHUNK)], buf, sem)

    def _scatter(buf, dbuf):
        pltpu.sync_copy(buf, acc_sh.at[dbuf], add=True)

    _gather(0, rows0_v, sem0).start()
    _dstcp(0, dst0_v, semd0).start()
    pltpu.sync_copy(zrows, acc_sh.at[pl.ds(sid * RPS, RPS)])

    @pl.when(sid == 0)
    def _zero_rem():
        pltpu.sync_copy(zrows.at[pl.ds(0, REMR)],
                        acc_sh.at[pl.ds(NS * RPS, REMR)])
    plsc.subcore_barrier()


    def _step(k, carry):
        c0 = 2 * k
        _gather(c0, rows0_v, sem0).wait()
        _dstcp(c0, dst0_v, semd0).wait()
        _gather(c0 + 1, rows1_v, sem1).start()
        _dstcp(c0 + 1, dst1_v, semd1).start()
        _scatter(rows0_v, dst0_v)

        @pl.when(c0 + 2 < NSTEPS)
        def _pref():
            _gather(c0 + 2, rows0_v, sem0).start()
            _dstcp(c0 + 2, dst0_v, semd0).start()
        _gather(c0 + 1, rows1_v, sem1).wait()
        _dstcp(c0 + 1, dst1_v, semd1).wait()
        _scatter(rows1_v, dst1_v)
        return carry
    lax.fori_loop(0, NSTEPS // 2, _step, 0)

    rsl = pl.ds(0, REME)
    pltpu.sync_copy(dsts.at[pl.ds(ebase + NSTEPS * CHUNK, REME)], dstr_v)
    pltpu.async_copy(table.at[gidx_v.at[pl.ds(NSTEPS * CHUNK, REME)]],
                     rows0_v.at[rsl], sem0).wait()
    pltpu.sync_copy(rows0_v.at[rsl], acc_sh.at[dstr_v], add=True)

    plsc.subcore_barrier()
    pltpu.sync_copy(acc_sh.at[pl.ds(sid * RPS, RPS)],
                    out.at[cid, pl.ds(sid * RPS, RPS)])

    @pl.when(sid == 0)
    def _out_rem():
        pltpu.sync_copy(acc_sh.at[pl.ds(NS * RPS, REMR)],
                        out.at[cid, pl.ds(NS * RPS, REMR)])


def _sc_agg(table, gidx, dsts, zrows):
    mesh = plsc.VectorSubcoreMesh(core_axis_name="c", subcore_axis_name="s")
    f = functools.partial(
        pl.kernel,
        mesh=mesh,
        out_type=jax.ShapeDtypeStruct((NC, N, D), jnp.float32),
        scratch_types=[
            pltpu.VMEM((EPW,), jnp.int32),
            pltpu.VMEM((CHUNK,), jnp.int32),
            pltpu.VMEM((CHUNK,), jnp.int32),
            pltpu.VMEM((REME,), jnp.int32),
            pltpu.VMEM((CHUNK, D), jnp.float32),
            pltpu.VMEM((CHUNK, D), jnp.float32),
            pltpu.VMEM_SHARED((N, D), jnp.float32),
            pltpu.SemaphoreType.DMA,
            pltpu.SemaphoreType.DMA,
            pltpu.SemaphoreType.DMA,
            pltpu.SemaphoreType.DMA,
        ],
    )(_sc_agg_body)
    return f(table, gidx, dsts, zrows)


BM = 2000
NB = N // BM


def _elu(x):
    return jnp.where(x > 0, x, jnp.exp(jnp.minimum(x, 0.0)) - 1.0)


def _mm_body(x_ref, w_ref, o_ref):
    acc = jnp.dot(x_ref[...], w_ref[...], preferred_element_type=jnp.float32)
    for s in range(S):
        o_ref[s] = acc[:, s * D:(s + 1) * D]


def _mm_table(h, Wr):
    return pl.pallas_call(
        _mm_body,
        grid=(NB,),
        in_specs=[
            pl.BlockSpec((BM, D), lambda i: (i, 0)),
            pl.BlockSpec((D, S * D), lambda i: (0, 0)),
        ],
        out_specs=pl.BlockSpec((S, BM, D), lambda i: (0, i, 0)),
        out_shape=jax.ShapeDtypeStruct((S, N, D), jnp.float32),
    )(h, Wr)


def _bn_mm_body(parts_ref, b_ref, g_ref, be_ref, w_ref, o_ref,
                stat_ref, h_ref):
    p = pl.program_id(0)
    i = pl.program_id(1)
    rows = pl.ds(i * BM, BM)

    @pl.when(p == 0)
    def _acc():
        a = parts_ref[0] + parts_ref[1] + b_ref[...]
        h_ref[rows, :] = a

        @pl.when(i == 0)
        def _init():
            stat_ref[...] = jnp.zeros_like(stat_ref)
        stat_ref[0:1, :] += jnp.sum(a, axis=0, keepdims=True)
        stat_ref[1:2, :] += jnp.sum(a * a, axis=0, keepdims=True)

    @pl.when(p == 1)
    def _norm_mm():
        m = stat_ref[0:1, :] / N
        v = stat_ref[1:2, :] / N - m * m
        h = _elu(g_ref[...] * (h_ref[rows, :] - m) * lax.rsqrt(v + 1e-5)
                 + be_ref[...])
        h_ref[rows, :] = h
        acc = jnp.dot(h, w_ref[...], preferred_element_type=jnp.float32)
        for s in range(S):
            o_ref[s] = acc[:, s * D:(s + 1) * D]


def _bn_mm(parts, b, g, be, Wr):
    vspec = pl.BlockSpec((1, D), lambda p, i: (0, 0))
    return pl.pallas_call(
        _bn_mm_body,
        grid=(2, NB),
        in_specs=[
            pl.BlockSpec((NC, BM, D),
                         lambda p, i: (0, jnp.where(p == 0, i, 0), 0)),
            vspec, vspec, vspec,
            pl.BlockSpec((D, S * D), lambda p, i: (0, 0)),
        ],
        out_specs=pl.BlockSpec(
            (S, BM, D), lambda p, i: (0, jnp.where(p == 1, i, 0), 0)),
        out_shape=jax.ShapeDtypeStruct((S, N, D), jnp.float32),
        scratch_shapes=[
            pltpu.VMEM((2, D), jnp.float32),
            pltpu.VMEM((N, D), jnp.float32),
        ],
    )(parts, b.reshape(1, D), g.reshape(1, D), be.reshape(1, D), Wr)


BF = 2000
NBF = N // BF


def _final_body(parts_ref, b2_ref, g2_ref, be2_ref, x_ref, w3_ref, b3_ref,
                g3_ref, be3_ref, o_ref, s2_ref, s3_ref, y_ref):
    p = pl.program_id(0)
    i = pl.program_id(1)
    rows = pl.ds(i * BF, BF)

    @pl.when(p == 0)
    def _acc2():
        a = parts_ref[0] + parts_ref[1] + b2_ref[...]
        y_ref[rows, :] = a

        @pl.when(i == 0)
        def _init():
            s2_ref[...] = jnp.zeros_like(s2_ref)
            s3_ref[...] = jnp.zeros_like(s3_ref)
        s2_ref[0:1, :] += jnp.sum(a, axis=0, keepdims=True)
        s2_ref[1:2, :] += jnp.sum(a * a, axis=0, keepdims=True)

    @pl.when(p == 1)
    def _mk_y():
        m = s2_ref[0:1, :] / N
        v = s2_ref[1:2, :] / N - m * m
        h2 = _elu(g2_ref[...] * (y_ref[rows, :] - m) * lax.rsqrt(v + 1e-5)
                  + be2_ref[...])
        y = h2 + jnp.dot(x_ref[...], w3_ref[...],
                         preferred_element_type=jnp.float32) + b3_ref[...]
        y_ref[rows, :] = y
        s3_ref[0:1, :] += jnp.sum(y, axis=0, keepdims=True)
        s3_ref[1:2, :] += jnp.sum(y * y, axis=0, keepdims=True)

    @pl.when(p == 2)
    def _apply():
        m = s3_ref[0:1, :] / N
        v = s3_ref[1:2, :] / N - m * m
        o_ref[...] = _elu(g3_ref[...] * (y_ref[rows, :] - m)
                          * lax.rsqrt(v + 1e-5) + be3_ref[...])


def _final(parts, b2, g2, be2, x, W3, b3, g3, be3):
    vecs = [v.reshape(1, D) for v in (b2, g2, be2, b3, g3, be3)]
    vspec = pl.BlockSpec((1, D), lambda p, i: (0, 0))
    return pl.pallas_call(
        _final_body,
        grid=(3, NBF),
        in_specs=[
            pl.BlockSpec((NC, BF, D),
                         lambda p, i: (0, jnp.where(p == 0, i, 0), 0)),
            vspec, vspec, vspec,
            pl.BlockSpec((BF, D), lambda p, i: (jnp.where(p == 1, i, 0), 0)),
            pl.BlockSpec((D, D), lambda p, i: (0, 0)),
            vspec, vspec, vspec,
        ],
        out_specs=pl.BlockSpec(
            (BF, D), lambda p, i: (jnp.where(p == 2, i, 0), 0)),
        out_shape=jax.ShapeDtypeStruct((N, D), jnp.float32),
        scratch_shapes=[
            pltpu.VMEM((2, D), jnp.float32),
            pltpu.VMEM((2, D), jnp.float32),
            pltpu.VMEM((N, D), jnp.float32),
        ],
    )(parts, vecs[0], vecs[1], vecs[2], x, W3, vecs[3], vecs[4], vecs[5])


def kernel(x, edge_index, selections, W1, b1, g1, be1, W2, b2, g2, be2,
           W3, b3, g3, be3):
    src = edge_index[0].astype(jnp.int32)
    dst = edge_index[1].astype(jnp.int32)
    sel = selections.astype(jnp.int32)
    gidx = sel * N + src

    zrows = jnp.zeros((RPS, D), jnp.float32)

    Wr1 = W1.transpose(1, 0, 2).reshape(D, S * D)
    Wr2 = W2.transpose(1, 0, 2).reshape(D, S * D)

    table1 = _mm_table(x, Wr1).reshape(S * N, D)
    parts1 = _sc_agg(table1, gidx, dst, zrows)
    table2 = _bn_mm(parts1, b1, g1, be1, Wr2).reshape(S * N, D)
    parts2 = _sc_agg(table2, gidx, dst, zrows)
    return _final(parts2, b2, g2, be2, x, W3, b3, g3, be3)
